# baseline (speedup 1.0000x reference)
import functools

import jax
import jax.numpy as jnp
from jax import lax
from jax.experimental import pallas as pl
from jax.experimental.pallas import tpu as pltpu
from jax.experimental.pallas import tpu_sc as plsc

NUM_CLASSES = 1000
HIDDEN_SIZE = 128
DROPOUT_PROB = 0.1
BATCH = 16384

_NC = 2
_NS = 16
_NW = _NC * _NS
_B_PER_W = BATCH // _NW
_CHUNK = 128
_N_CHUNKS = _B_PER_W // _CHUNK


def _embed_body(table_hbm, idx_hbm, out_hbm, idx_v, rows_v, isem, gsem):
    wid = lax.axis_index("s") * _NC + lax.axis_index("c")
    base = wid * _B_PER_W
    idx_copies = []
    for c in range(_N_CHUNKS):
        idx_copies.append(
            pltpu.async_copy(
                idx_hbm.at[pl.ds(base + c * _CHUNK, _CHUNK)],
                idx_v.at[pl.ds(c * _CHUNK, _CHUNK)],
                isem.at[c],
            )
        )
    gathers = []
    for c in range(_N_CHUNKS):
        idx_copies[c].wait()
        gathers.append(
            pltpu.async_copy(
                table_hbm.at[idx_v.at[pl.ds(c * _CHUNK, _CHUNK)]],
                rows_v.at[pl.ds(c * _CHUNK, _CHUNK)],
                gsem,
            )
        )
    for g in gathers:
        g.wait()
    pltpu.sync_copy(rows_v, out_hbm.at[pl.ds(base, _B_PER_W)])


@jax.jit
def _embed(table_i32, idx):
    mesh = plsc.VectorSubcoreMesh(core_axis_name="c", subcore_axis_name="s")
    return pl.kernel(
        _embed_body,
        mesh=mesh,
        compiler_params=pltpu.CompilerParams(use_tc_tiling_on_sc=False),
        out_type=jax.ShapeDtypeStruct((BATCH, HIDDEN_SIZE // 2), jnp.int32),
        scratch_types=[
            pltpu.VMEM((_B_PER_W,), jnp.int32),
            pltpu.VMEM((_B_PER_W, HIDDEN_SIZE // 2), jnp.int32),
            pltpu.SemaphoreType.DMA((_N_CHUNKS,)),
            pltpu.SemaphoreType.DMA,
        ],
    )(table_i32, idx)


def kernel(labels, train, table):
    use_drop = jnp.logical_and(jnp.asarray(train) != 0, DROPOUT_PROB > 0.0)
    drop_key = jax.random.key(1)
    drop_ids = jax.random.uniform(drop_key, (labels.shape[0],)) < DROPOUT_PROB
    idx = jnp.where(jnp.logical_and(use_drop, drop_ids), NUM_CLASSES, labels)
    table_i32 = lax.bitcast_convert_type(
        table.astype(jnp.bfloat16).reshape(NUM_CLASSES + 1, HIDDEN_SIZE // 2, 2),
        jnp.int32,
    )
    out_i32 = _embed(table_i32, idx.astype(jnp.int32))
    return lax.bitcast_convert_type(out_i32, jnp.bfloat16).reshape(
        BATCH, HIDDEN_SIZE
    ).astype(jnp.float32)


# speedup vs baseline: 2.1457x; 2.1457x over previous
import functools

import jax
import jax.numpy as jnp
from jax import lax
from jax.experimental import pallas as pl
from jax.experimental.pallas import tpu as pltpu
from jax.experimental.pallas import tpu_sc as plsc

NUM_CLASSES = 1000
HIDDEN_SIZE = 128
DROPOUT_PROB = 0.1
BATCH = 16384

_NC = 2
_NS = 16
_NW = _NC * _NS

_S_SC = 8192
_B_PER_W = _S_SC // _NW
_CHUNK = 128
_N_CHUNKS = _B_PER_W // _CHUNK

_TC_BLK = 512
_VPAD = 1024


def _sc_body(table_hbm, idx_hbm, out_hbm, idx_v, rows_v, isem, gsem):
    wid = lax.axis_index("s") * _NC + lax.axis_index("c")
    base = wid * _B_PER_W
    idx_copies = []
    for c in range(_N_CHUNKS):
        idx_copies.append(
            pltpu.async_copy(
                idx_hbm.at[pl.ds(base + c * _CHUNK, _CHUNK)],
                idx_v.at[pl.ds(c * _CHUNK, _CHUNK)],
                isem.at[c],
            )
        )
    gathers = []
    for c in range(_N_CHUNKS):
        idx_copies[c].wait()
        gathers.append(
            pltpu.async_copy(
                table_hbm.at[idx_v.at[pl.ds(c * _CHUNK, _CHUNK)]],
                rows_v.at[pl.ds(c * _CHUNK, _CHUNK)],
                gsem,
            )
        )
    for g in gathers:
        g.wait()
    pltpu.sync_copy(rows_v, out_hbm.at[pl.ds(base, _B_PER_W)])


def _tc_body(labels_ref, table_ref, out_ref):
    lbl = labels_ref[...]
    oh = lbl[:, None] == lax.broadcasted_iota(jnp.int32, (_TC_BLK, _VPAD), 1)
    out_ref[...] = jnp.dot(
        oh.astype(jnp.bfloat16),
        table_ref[...],
        preferred_element_type=jnp.float32,
    )


@jax.jit
def _embed(table, table_pad_bf16, idx):
    tc_out = pl.pallas_call(
        _tc_body,
        grid=((BATCH - _S_SC) // _TC_BLK,),
        in_specs=[
            pl.BlockSpec((_TC_BLK,), lambda i: (i + _S_SC // _TC_BLK,)),
            pl.BlockSpec((_VPAD, HIDDEN_SIZE), lambda i: (0, 0)),
        ],
        out_specs=pl.BlockSpec((_TC_BLK, HIDDEN_SIZE), lambda i: (i + _S_SC // _TC_BLK, 0)),
        out_shape=jax.ShapeDtypeStruct((BATCH, HIDDEN_SIZE), jnp.float32),
    )(idx, table_pad_bf16)

    out_ref = jax.new_ref(tc_out)
    mesh = plsc.VectorSubcoreMesh(core_axis_name="c", subcore_axis_name="s")
    pl.kernel(
        _sc_body,
        mesh=mesh,
        out_type=(),
        scratch_types=[
            pltpu.VMEM((_B_PER_W,), jnp.int32),
            pltpu.VMEM((_B_PER_W, HIDDEN_SIZE), jnp.float32),
            pltpu.SemaphoreType.DMA((_N_CHUNKS,)),
            pltpu.SemaphoreType.DMA,
        ],
    )(table, idx, out_ref)
    return out_ref[...]


def kernel(labels, train, table):
    use_drop = jnp.logical_and(jnp.asarray(train) != 0, DROPOUT_PROB > 0.0)
    drop_key = jax.random.key(1)
    drop_ids = jax.random.uniform(drop_key, (labels.shape[0],)) < DROPOUT_PROB
    idx = jnp.where(jnp.logical_and(use_drop, drop_ids), NUM_CLASSES, labels)
    tpad = jnp.pad(table, ((0, _VPAD - NUM_CLASSES - 1), (0, 0))).astype(
        jnp.bfloat16
    )
    return _embed(table, tpad, idx.astype(jnp.int32))


# speedup vs baseline: 2.8283x; 1.3181x over previous
import functools

import jax
import jax.numpy as jnp
from jax import lax
from jax.experimental import pallas as pl
from jax.experimental.pallas import tpu as pltpu
from jax.experimental.pallas import tpu_sc as plsc

NUM_CLASSES = 1000
HIDDEN_SIZE = 128
DROPOUT_PROB = 0.1
BATCH = 16384

_NC = 2
_NS = 16
_NW = _NC * _NS
_B_PER_W = BATCH // _NW
_CHUNK = 128
_N_CHUNKS = _B_PER_W // _CHUNK


def _embed_body(table_hbm, idx_hbm, out_hbm, idx_v, rows_v, isem, gsem):
    wid = lax.axis_index("s") * _NC + lax.axis_index("c")
    base = wid * _B_PER_W
    idx_copies = []
    for c in range(_N_CHUNKS):
        idx_copies.append(
            pltpu.async_copy(
                idx_hbm.at[pl.ds(base + c * _CHUNK, _CHUNK)],
                idx_v.at[pl.ds(c * _CHUNK, _CHUNK)],
                isem.at[c],
            )
        )
    gathers = []
    for c in range(_N_CHUNKS):
        idx_copies[c].wait()
        gathers.append(
            pltpu.async_copy(
                table_hbm.at[idx_v.at[pl.ds(c * _CHUNK, _CHUNK)]],
                rows_v.at[pl.ds(c * _CHUNK, _CHUNK)],
                gsem,
            )
        )
    for g in gathers:
        g.wait()
    pltpu.sync_copy(rows_v, out_hbm.at[pl.ds(base, _B_PER_W)])


@jax.jit
def _embed(table, idx):
    mesh = plsc.VectorSubcoreMesh(core_axis_name="c", subcore_axis_name="s")
    return pl.kernel(
        _embed_body,
        mesh=mesh,
        out_type=jax.ShapeDtypeStruct((BATCH, HIDDEN_SIZE), jnp.float32),
        scratch_types=[
            pltpu.VMEM((_B_PER_W,), jnp.int32),
            pltpu.VMEM((_B_PER_W, HIDDEN_SIZE), jnp.float32),
            pltpu.SemaphoreType.DMA((_N_CHUNKS,)),
            pltpu.SemaphoreType.DMA,
        ],
    )(table, idx)


def _remap_dropped(labels):
    drop_key = jax.random.key(1)
    drop_ids = jax.random.uniform(drop_key, (labels.shape[0],)) < DROPOUT_PROB
    return jnp.where(drop_ids, NUM_CLASSES, labels).astype(jnp.int32)


def kernel(labels, train, table):
    idx = lax.cond(
        jnp.asarray(train) != 0,
        _remap_dropped,
        lambda l: l.astype(jnp.int32),
        labels,
    )
    return _embed(table, idx)
